# baseline (speedup 1.0000x reference)
import functools

import jax
import jax.numpy as jnp
from jax import lax
from jax.experimental import pallas as pl
from jax.experimental.pallas import tpu as pltpu
from jax.experimental.pallas import tpu_sc as plsc

N_ROOM = 10000
N_RV = 1000
E_RR = 320000
E_POOL = 10000

NC = 2
NS = 16

TPB = 640
N_ACC2 = NS * TPB
ACC_R = TPB + 8
S = 2048
CAP = S + 128
NPO = 1280
CNT_W = 16

RR_STRIPS = 80
PP_STRIPS = 3

_f32 = jnp.float32


def _pad_edges(src, dst, nstrips):
    e_pad = 2 * nstrips * S
    e = src.shape[0]
    src_p = jnp.concatenate([src, jnp.zeros((e_pad - e,), jnp.int32)])
    dst_p = jnp.concatenate([dst, jnp.full((e_pad - e,), N_ROOM, jnp.int32)])
    return src_p, dst_p


def _sc_segsums(width, passes, with_counts):
    np_ = len(passes)
    inputs = []
    for tab, src1, dst1, nstrips, orows in passes:
        inputs += [tab, src1, dst1]
    inputs.append(jnp.zeros((ACC_R, width), _f32))
    if with_counts:
        inputs.append(jnp.zeros((ACC_R, CNT_W), _f32))

    out_type = [jax.ShapeDtypeStruct((p[4], width), _f32)
                for p in passes for _ in range(NC)]
    if with_counts:
        out_type += [jax.ShapeDtypeStruct((p[4], CNT_W), _f32)
                     for p in passes for _ in range(NC)]

    scratch = [
        pltpu.VMEM((S,), jnp.int32),
        pltpu.VMEM((S,), jnp.int32),
        pltpu.VMEM((CAP,), jnp.int32),
        pltpu.VMEM((CAP // 128, 128), jnp.int32),
        pltpu.VMEM((128, width), _f32),
        pltpu.VMEM((ACC_R, width), _f32),
    ]
    if with_counts:
        scratch.append(pltpu.VMEM((ACC_R, CNT_W), _f32))
    scratch.append(pltpu.SemaphoreType.DMA)

    def body(*refs):
        it = iter(refs)
        tabs, srcs, dsts = [], [], []
        for _ in range(np_):
            tabs.append(next(it)); srcs.append(next(it)); dsts.append(next(it))
        zw = next(it)
        if with_counts:
            zc = next(it)
        outs = [(next(it), next(it)) for _ in range(np_)]
        couts = [(next(it), next(it)) for _ in range(np_)] if with_counts else []
        sbuf = next(it); dbuf = next(it); srcm = next(it); dstm2 = next(it)
        rows = next(it); acc = next(it)
        if with_counts:
            cacc = next(it)
        sem = next(it)

        cid = lax.axis_index("c")
        sid = lax.axis_index("s")
        lo = sid * TPB
        iota16 = lax.iota(jnp.int32, 16)
        all_on = jnp.ones((16,), jnp.bool_)
        pad_src = jnp.zeros((16,), jnp.int32)
        pad_dst = jnp.full((16,), TPB, jnp.int32)
        one16 = jnp.full((16,), 1.0, _f32)
        use_counts = with_counts

        for p in range(np_):
            tab = tabs[p]
            src1 = srcs[p]
            dst1 = dsts[p]
            nstrips = passes[p][3]
            orows = passes[p][4]
            base = cid * (nstrips * S)

            pltpu.sync_copy(zw, acc)
            if use_counts:
                pltpu.sync_copy(zc, cacc)

            @pl.loop(0, nstrips)
            def _(si):
                off_e = base + si * S
                pltpu.sync_copy(src1.at[pl.ds(off_e, S)], sbuf)
                pltpu.sync_copy(dst1.at[pl.ds(off_e, S)], dbuf)

                def scan_body(v, m):
                    dv = dbuf[pl.ds(v * 16, 16)]
                    sv = sbuf[pl.ds(v * 16, 16)]
                    mask = (dv >= lo) & (dv < lo + TPB)
                    mi = mask.astype(jnp.int32)
                    pos = m + plsc.cumsum(mi) - 1
                    plsc.store_scatter(srcm, [pos], sv, mask=mask)
                    plsc.store_scatter(
                        dstm2,
                        [lax.shift_right_logical(pos, 7),
                         lax.bitwise_and(pos, 127)],
                        dv - lo, mask=mask)
                    return m + jnp.sum(mi)

                m = pl.loop(0, S // 16, init_carry=jnp.int32(0))(scan_body)

                for k in range(8):
                    padpos = m + k * 16 + iota16
                    plsc.store_scatter(srcm, [padpos], pad_src, mask=all_on)
                    plsc.store_scatter(
                        dstm2,
                        [lax.shift_right_logical(padpos, 7),
                         lax.bitwise_and(padpos, 127)],
                        pad_dst, mask=all_on)

                nch = lax.div(m + 127, jnp.int32(128))

                @pl.loop(0, nch)
                def _(c):
                    off = pl.multiple_of(c * 128, 128)
                    pltpu.async_copy(
                        tab.at[srcm.at[pl.ds(off, 128)]], rows, sem).wait()

                    @pl.loop(0, 8)
                    def _(t):
                        dvec = dstm2[c, pl.ds(t * 16, 16)]
                        for lane in range(16):
                            d = dvec[lane]
                            j = t * 16 + lane
                            for k2 in range(width // 16):
                                plsc.addupdate(
                                    acc.at[d, pl.ds(k2 * 16, 16)],
                                    rows[j, pl.ds(k2 * 16, 16)])
                            if use_counts:
                                plsc.addupdate(cacc.at[d, pl.ds(0, CNT_W)],
                                               one16)

            o0, o1 = outs[p]
            for core, oref in enumerate((o0, o1)):
                @pl.when((cid == core) & (lo < orows))
                def _(oref=oref, core=core):
                    pltpu.sync_copy(acc.at[pl.ds(0, TPB)],
                                    oref.at[pl.ds(lo, TPB)])
                    if use_counts:
                        pltpu.sync_copy(cacc.at[pl.ds(0, TPB)],
                                        couts[p][core].at[pl.ds(lo, TPB)])

    mesh = plsc.VectorSubcoreMesh(core_axis_name="c", subcore_axis_name="s")
    fn = pl.kernel(body, out_type=out_type, mesh=mesh, scratch_types=scratch,
                   compiler_params=pltpu.CompilerParams(
                       use_tc_tiling_on_sc=False, needs_layout_passes=False))
    res = fn(*inputs)
    if not isinstance(res, (list, tuple)):
        res = [res]
    sums = [(res[2 * p], res[2 * p + 1]) for p in range(np_)]
    if with_counts:
        counts = [(res[2 * np_ + 2 * p], res[2 * np_ + 2 * p + 1])
                  for p in range(np_)]
    else:
        counts = [None] * np_
    return sums, counts


def _mm_multi(x, ws, bs, bm):
    rows, k = x.shape
    grid = rows // bm
    nw = len(ws)

    def bodyf(*refs):
        xr = refs[0]
        wr = refs[1:1 + nw]
        br = refs[1 + nw:1 + 2 * nw]
        outs = refs[1 + 2 * nw:]
        xv = xr[...]
        for i in range(nw):
            outs[i][...] = jnp.dot(xv, wr[i][...],
                                   preferred_element_type=_f32) + br[i][...]

    in_specs = [pl.BlockSpec((bm, k), lambda i: (i, 0))]
    in_specs += [pl.BlockSpec(w.shape, lambda i: (0, 0)) for w in ws]
    in_specs += [pl.BlockSpec((1, w.shape[1]), lambda i: (0, 0)) for w in ws]
    out_specs = [pl.BlockSpec((bm, w.shape[1]), lambda i: (i, 0)) for w in ws]
    out_shape = [jax.ShapeDtypeStruct((rows, w.shape[1]), _f32) for w in ws]
    res = pl.pallas_call(
        bodyf, grid=(grid,), in_specs=in_specs, out_specs=out_specs,
        out_shape=out_shape,
    )(x, *ws, *[b.reshape(1, -1) for b in bs])
    return list(res) if isinstance(res, (list, tuple)) else [res]


def _combine_room(s_a, c_a, s_b, c_b, z, ws, bs, relu, bm):
    rows, w_in = z.shape
    grid = rows // bm
    nw = len(ws)

    def bodyf(*refs):
        (sa0, sa1, ca0, ca1, sb0, sb1, cb0, cb1, zr) = refs[:9]
        wr = refs[9:9 + nw]
        br = refs[9 + nw:9 + 2 * nw]
        outs = refs[9 + 2 * nw:]
        cnt_a = jnp.maximum(ca0[:, :1] + ca1[:, :1], 1.0)
        cnt_b = jnp.maximum(cb0[:, :1] + cb1[:, :1], 1.0)
        x = (sa0[...] + sa1[...]) / cnt_a + (sb0[...] + sb1[...]) / cnt_b + zr[...]
        if relu:
            x = jnp.maximum(x, 0.0)
        if nw == 0:
            outs[0][...] = x
        else:
            for i in range(nw):
                outs[i][...] = jnp.dot(x, wr[i][...],
                                       preferred_element_type=_f32) + br[i][...]

    in_specs = [
        pl.BlockSpec((bm, w_in), lambda i: (i, 0)),
        pl.BlockSpec((bm, w_in), lambda i: (i, 0)),
        pl.BlockSpec((bm, CNT_W), lambda i: (i, 0)),
        pl.BlockSpec((bm, CNT_W), lambda i: (i, 0)),
        pl.BlockSpec((bm, w_in), lambda i: (i, 0)),
        pl.BlockSpec((bm, w_in), lambda i: (i, 0)),
        pl.BlockSpec((bm, CNT_W), lambda i: (i, 0)),
        pl.BlockSpec((bm, CNT_W), lambda i: (i, 0)),
        pl.BlockSpec((bm, w_in), lambda i: (i, 0)),
    ]
    in_specs += [pl.BlockSpec(w.shape, lambda i: (0, 0)) for w in ws]
    in_specs += [pl.BlockSpec((1, w.shape[1]), lambda i: (0, 0)) for w in ws]
    if nw == 0:
        out_specs = [pl.BlockSpec((bm, w_in), lambda i: (i, 0))]
        out_shape = [jax.ShapeDtypeStruct((rows, w_in), _f32)]
    else:
        out_specs = [pl.BlockSpec((bm, w.shape[1]), lambda i: (i, 0)) for w in ws]
        out_shape = [jax.ShapeDtypeStruct((rows, w.shape[1]), _f32) for w in ws]
    res = pl.pallas_call(
        bodyf, grid=(grid,), in_specs=in_specs, out_specs=out_specs,
        out_shape=out_shape,
    )(s_a[0], s_a[1], c_a[0], c_a[1], s_b[0], s_b[1], c_b[0], c_b[1], z,
      *ws, *[b.reshape(1, -1) for b in bs])
    return list(res) if isinstance(res, (list, tuple)) else [res]


def _combine_rv(s, c, z, ws, bs, relu):
    w_in = z.shape[1]
    nw = len(ws)

    def bodyf(*refs):
        s0, s1, c0, c1, zr = refs[:5]
        wr = refs[5:5 + nw]
        br = refs[5 + nw:5 + 2 * nw]
        outs = refs[5 + 2 * nw:]
        ssum = (s0[...] + s1[...])[:N_RV]
        cnt = jnp.maximum((c0[...] + c1[...])[:N_RV, :1], 1.0)
        x = ssum / cnt + zr[...]
        if relu:
            x = jnp.maximum(x, 0.0)
        if nw == 0:
            outs[0][...] = x
        else:
            for i in range(nw):
                outs[i][...] = jnp.dot(x, wr[i][...],
                                       preferred_element_type=_f32) + br[i][...]

    in_specs = [
        pl.BlockSpec((NPO, w_in), lambda i: (0, 0)),
        pl.BlockSpec((NPO, w_in), lambda i: (0, 0)),
        pl.BlockSpec((NPO, CNT_W), lambda i: (0, 0)),
        pl.BlockSpec((NPO, CNT_W), lambda i: (0, 0)),
        pl.BlockSpec((N_RV, w_in), lambda i: (0, 0)),
    ]
    in_specs += [pl.BlockSpec(w.shape, lambda i: (0, 0)) for w in ws]
    in_specs += [pl.BlockSpec((1, w.shape[1]), lambda i: (0, 0)) for w in ws]
    if nw == 0:
        out_specs = [pl.BlockSpec((N_RV, w_in), lambda i: (0, 0))]
        out_shape = [jax.ShapeDtypeStruct((N_RV, w_in), _f32)]
    else:
        out_specs = [pl.BlockSpec((N_RV, w.shape[1]), lambda i: (0, 0)) for w in ws]
        out_shape = [jax.ShapeDtypeStruct((N_RV, w.shape[1]), _f32) for w in ws]
    res = pl.pallas_call(
        bodyf, grid=(1,), in_specs=in_specs, out_specs=out_specs,
        out_shape=out_shape,
    )(s[0], s[1], c[0], c[1], z, *ws, *[b.reshape(1, -1) for b in bs])
    return list(res) if isinstance(res, (list, tuple)) else [res]


def kernel(x_room, x_room_virtual, edge_index_rr, edge_index_r_rv, edge_index_rv_r,
           Wn_0_rr, Wr_0_rr, b_0_rr, Wn_0_r_rv, Wr_0_r_rv, b_0_r_rv, Wn_0_rv_r, Wr_0_rv_r, b_0_rv_r,
           Wn_1_rr, Wr_1_rr, b_1_rr, Wn_1_r_rv, Wr_1_r_rv, b_1_r_rv, Wn_1_rv_r, Wr_1_rv_r, b_1_rv_r,
           Wn_2_rr, Wr_2_rr, b_2_rr, Wn_2_r_rv, Wr_2_r_rv, b_2_r_rv, Wn_2_rv_r, Wr_2_rv_r, b_2_rv_r):
    BM = 1000

    srr2, drr2 = _pad_edges(edge_index_rr[0], edge_index_rr[1], RR_STRIPS)
    sprv2, dprv2 = _pad_edges(edge_index_r_rv[0], edge_index_r_rv[1], PP_STRIPS)
    srvr2, drvr2 = _pad_edges(edge_index_rv_r[0], edge_index_rv_r[1], PP_STRIPS)

    y_rr0, y_prv0, z_room0 = _mm_multi(
        x_room, [Wn_0_rr, Wn_0_r_rv, Wr_0_rr + Wr_0_rv_r],
        [jnp.zeros_like(b_0_rr), jnp.zeros_like(b_0_rr), b_0_rr + b_0_rv_r], BM)
    y_rvr0, z_rv0 = _mm_multi(
        x_room_virtual, [Wn_0_rv_r, Wr_0_r_rv],
        [jnp.zeros_like(b_0_rv_r), b_0_r_rv], N_RV)

    (s_rr0, s_rvr0, s_prv0), (c_rr, c_rvr, c_prv) = _sc_segsums(
        128,
        [(y_rr0, srr2, drr2, RR_STRIPS, N_ACC2),
         (y_rvr0, srvr2, drvr2, PP_STRIPS, N_ACC2),
         (y_prv0, sprv2, dprv2, PP_STRIPS, NPO)],
        with_counts=True)

    y_rr1, y_prv1, z_room1 = _combine_room(
        s_rr0, c_rr, s_rvr0, c_rvr, z_room0,
        [Wn_1_rr, Wn_1_r_rv, Wr_1_rr + Wr_1_rv_r],
        [jnp.zeros_like(b_1_rr), jnp.zeros_like(b_1_rr), b_1_rr + b_1_rv_r],
        relu=True, bm=BM)
    y_rvr1, z_rv1 = _combine_rv(
        s_prv0, c_prv, z_rv0, [Wn_1_rv_r, Wr_1_r_rv],
        [jnp.zeros_like(b_1_rv_r), b_1_r_rv], relu=True)

    (s_rr1, s_rvr1, s_prv1), _ = _sc_segsums(
        128,
        [(y_rr1, srr2, drr2, RR_STRIPS, N_ACC2),
         (y_rvr1, srvr2, drvr2, PP_STRIPS, N_ACC2),
         (y_prv1, sprv2, dprv2, PP_STRIPS, NPO)],
        with_counts=False)

    y_rr2, z_room2 = _combine_room(
        s_rr1, c_rr, s_rvr1, c_rvr, z_room1,
        [Wn_2_rr, Wr_2_rr + Wr_2_rv_r],
        [jnp.zeros_like(b_2_rr), b_2_rr + b_2_rv_r], relu=True, bm=BM)
    (y_rvr2,) = _combine_rv(
        s_prv1, c_prv, z_rv1, [Wn_2_rv_r], [jnp.zeros_like(b_2_rv_r)], relu=True)

    (s_rr2, s_rvr2), _ = _sc_segsums(
        32,
        [(y_rr2, srr2, drr2, RR_STRIPS, N_ACC2),
         (y_rvr2, srvr2, drvr2, PP_STRIPS, N_ACC2)],
        with_counts=False)

    (x3,) = _combine_room(s_rr2, c_rr, s_rvr2, c_rvr, z_room2, [], [],
                          relu=False, bm=BM)

    (s_pool,), _ = _sc_segsums(
        32, [(x3, sprv2, dprv2, PP_STRIPS, NPO)], with_counts=False)

    (out,) = _combine_rv(s_pool, c_prv, jnp.zeros((N_RV, 32), _f32), [], [],
                         relu=False)
    return out


# speedup vs baseline: 4.2122x; 4.2122x over previous
import jax
import jax.numpy as jnp
from jax import lax
from jax.experimental import pallas as pl
from jax.experimental.pallas import tpu as pltpu
from jax.experimental.pallas import tpu_sc as plsc

N_ROOM = 10000
N_RV = 1000
E_RR = 320000
E_POOL = 10000

NC = 2
NS = 16
NW = NC * NS
CH = 128

N_ACC = 10112
NPO = 1024
CNT_W = 16
G = 8

RR_NTC = 80
PP_NTC = 8

_f32 = jnp.float32


def _pad_edges(src, dst, ntc):
    e_pad = NW * ntc * CH
    e = src.shape[0]
    src_p = jnp.concatenate([src, jnp.zeros((e_pad - e,), jnp.int32)])
    dst_p = jnp.concatenate([dst, jnp.full((e_pad - e,), N_ROOM, jnp.int32)])
    return src_p.reshape(NW, ntc, CH), dst_p.reshape(NW, ntc, CH)


_SC_PARAMS = pltpu.CompilerParams(use_tc_tiling_on_sc=False)
_ZR = N_ACC // NS


def _sc_counts(passes):
    np_ = len(passes)
    inputs = [p[0] for p in passes]
    inputs.append(jnp.zeros((N_ACC, CNT_W), _f32))
    inputs.append(jnp.ones((CH, CNT_W), _f32))
    out_type = [jax.ShapeDtypeStruct((p[2], CNT_W), _f32)
                for p in passes for _ in range(NC)]
    scratch = [
        pltpu.VMEM((G, CH), jnp.int32),
        pltpu.VMEM((CH, CNT_W), _f32),
        pltpu.VMEM_SHARED((N_ACC, CNT_W), _f32),
        pltpu.SemaphoreType.DMA,
    ]

    def body(*refs):
        it = iter(refs)
        dsts = [next(it) for _ in range(np_)]
        zc = next(it)
        ones_h = next(it)
        couts = [(next(it), next(it)) for _ in range(np_)]
        didx = next(it)
        onesv = next(it)
        accc = next(it)
        sem = next(it)

        cid = lax.axis_index("c")
        sid = lax.axis_index("s")
        wid = cid * NS + sid
        pltpu.sync_copy(ones_h, onesv)

        for p in range(np_):
            ntc = passes[p][1]
            orows = passes[p][2]
            pltpu.sync_copy(zc.at[pl.ds(sid * _ZR, _ZR)],
                            accc.at[pl.ds(sid * _ZR, _ZR)])
            plsc.subcore_barrier()

            dstp = dsts[p]

            @pl.loop(0, ntc // G)
            def _(gi):
                pltpu.sync_copy(dstp.at[wid, pl.ds(gi * G, G)], didx)
                for i in range(G):
                    pltpu.async_copy(onesv, accc.at[didx.at[i]], sem, add=True)
                for i in range(G):
                    pltpu.make_async_copy(onesv, accc.at[didx.at[i]], sem).wait()

            plsc.subcore_barrier()
            r = orows // NS
            o0, o1 = couts[p]
            for core, oref in enumerate((o0, o1)):
                @pl.when(cid == core)
                def _(oref=oref):
                    pltpu.sync_copy(accc.at[pl.ds(sid * r, r)],
                                    oref.at[pl.ds(sid * r, r)])
            plsc.subcore_barrier()

    mesh = plsc.VectorSubcoreMesh(core_axis_name="c", subcore_axis_name="s")
    fn = pl.kernel(body, out_type=out_type, mesh=mesh, scratch_types=scratch,
                   compiler_params=_SC_PARAMS)
    res = fn(*inputs)
    return [(res[2 * p], res[2 * p + 1]) for p in range(np_)]


def _sc_segsums(width, passes):
    np_ = len(passes)
    inputs = []
    for tab, src3, dst3, ntc, orows in passes:
        inputs += [tab, src3, dst3]
    inputs.append(jnp.zeros((N_ACC, width), _f32))
    out_type = [jax.ShapeDtypeStruct((p[4], width), _f32)
                for p in passes for _ in range(NC)]
    scratch = [
        pltpu.VMEM((G, CH), jnp.int32),
        pltpu.VMEM((G, CH), jnp.int32),
        pltpu.VMEM((CH, width), _f32),
        pltpu.VMEM((CH, width), _f32),
        pltpu.VMEM_SHARED((N_ACC, width), _f32),
        pltpu.SemaphoreType.DMA,
        pltpu.SemaphoreType.DMA,
        pltpu.SemaphoreType.DMA,
        pltpu.SemaphoreType.DMA,
    ]

    def body(*refs):
        it = iter(refs)
        tabs, srcs, dsts = [], [], []
        for _ in range(np_):
            tabs.append(next(it)); srcs.append(next(it)); dsts.append(next(it))
        zw = next(it)
        outs = [(next(it), next(it)) for _ in range(np_)]
        sidx = next(it); didx = next(it)
        rowsA = next(it); rowsB = next(it)
        acc = next(it)
        gsemA = next(it); gsemB = next(it)
        ssemA = next(it); ssemB = next(it)

        cid = lax.axis_index("c")
        sid = lax.axis_index("s")
        wid = cid * NS + sid
        bufs = (rowsA, rowsB)
        gsems = (gsemA, gsemB)
        ssems = (ssemA, ssemB)

        for p in range(np_):
            ntc = passes[p][3]
            orows = passes[p][4]
            pltpu.sync_copy(zw.at[pl.ds(sid * _ZR, _ZR)],
                            acc.at[pl.ds(sid * _ZR, _ZR)])
            plsc.subcore_barrier()

            tab = tabs[p]
            srcp = srcs[p]
            dstp = dsts[p]

            @pl.loop(0, ntc // G)
            def _(gi):
                pltpu.sync_copy(srcp.at[wid, pl.ds(gi * G, G)], sidx)
                pltpu.sync_copy(dstp.at[wid, pl.ds(gi * G, G)], didx)
                pltpu.async_copy(tab.at[sidx.at[0]], bufs[0], gsems[0])
                for k in range(G):
                    b = k % 2
                    nb = (k + 1) % 2
                    pltpu.make_async_copy(tab.at[sidx.at[k]], bufs[b],
                                          gsems[b]).wait()
                    pltpu.async_copy(bufs[b], acc.at[didx.at[k]], ssems[b],
                                     add=True)
                    if k + 1 < G:
                        if k >= 1:
                            pltpu.make_async_copy(
                                bufs[nb], acc.at[didx.at[k - 1]],
                                ssems[nb]).wait()
                        pltpu.async_copy(tab.at[sidx.at[k + 1]], bufs[nb],
                                         gsems[nb])
                pltpu.make_async_copy(bufs[(G - 2) % 2],
                                      acc.at[didx.at[G - 2]],
                                      ssems[(G - 2) % 2]).wait()
                pltpu.make_async_copy(bufs[(G - 1) % 2],
                                      acc.at[didx.at[G - 1]],
                                      ssems[(G - 1) % 2]).wait()

            plsc.subcore_barrier()
            r = orows // NS
            o0, o1 = outs[p]
            for core, oref in enumerate((o0, o1)):
                @pl.when(cid == core)
                def _(oref=oref):
                    pltpu.sync_copy(acc.at[pl.ds(sid * r, r)],
                                    oref.at[pl.ds(sid * r, r)])
            plsc.subcore_barrier()

    mesh = plsc.VectorSubcoreMesh(core_axis_name="c", subcore_axis_name="s")
    fn = pl.kernel(body, out_type=out_type, mesh=mesh, scratch_types=scratch,
                   compiler_params=_SC_PARAMS)
    res = fn(*inputs)
    if not isinstance(res, (list, tuple)):
        res = [res]
    return [(res[2 * p], res[2 * p + 1]) for p in range(np_)]


def _mm_multi(x, ws, bs, bm):
    rows, k = x.shape
    grid = rows // bm
    nw = len(ws)

    def bodyf(*refs):
        xr = refs[0]
        wr = refs[1:1 + nw]
        br = refs[1 + nw:1 + 2 * nw]
        outs = refs[1 + 2 * nw:]
        xv = xr[...]
        for i in range(nw):
            outs[i][...] = jnp.dot(xv, wr[i][...],
                                   preferred_element_type=_f32) + br[i][...]

    in_specs = [pl.BlockSpec((bm, k), lambda i: (i, 0))]
    in_specs += [pl.BlockSpec(w.shape, lambda i: (0, 0)) for w in ws]
    in_specs += [pl.BlockSpec((1, w.shape[1]), lambda i: (0, 0)) for w in ws]
    out_specs = [pl.BlockSpec((bm, w.shape[1]), lambda i: (i, 0)) for w in ws]
    out_shape = [jax.ShapeDtypeStruct((rows, w.shape[1]), _f32) for w in ws]
    res = pl.pallas_call(
        bodyf, grid=(grid,), in_specs=in_specs, out_specs=out_specs,
        out_shape=out_shape,
    )(x, *ws, *[b.reshape(1, -1) for b in bs])
    return list(res) if isinstance(res, (list, tuple)) else [res]


def _combine_room(s_a, c_a, s_b, c_b, z, ws, bs, relu, bm):
    rows, w_in = z.shape
    grid = rows // bm
    nw = len(ws)

    def bodyf(*refs):
        (sa0, sa1, ca0, ca1, sb0, sb1, cb0, cb1, zr) = refs[:9]
        wr = refs[9:9 + nw]
        br = refs[9 + nw:9 + 2 * nw]
        outs = refs[9 + 2 * nw:]
        cnt_a = jnp.maximum(ca0[:, :1] + ca1[:, :1], 1.0)
        cnt_b = jnp.maximum(cb0[:, :1] + cb1[:, :1], 1.0)
        x = (sa0[...] + sa1[...]) / cnt_a + (sb0[...] + sb1[...]) / cnt_b + zr[...]
        if relu:
            x = jnp.maximum(x, 0.0)
        if nw == 0:
            outs[0][...] = x
        else:
            for i in range(nw):
                outs[i][...] = jnp.dot(x, wr[i][...],
                                       preferred_element_type=_f32) + br[i][...]

    in_specs = [
        pl.BlockSpec((bm, w_in), lambda i: (i, 0)),
        pl.BlockSpec((bm, w_in), lambda i: (i, 0)),
        pl.BlockSpec((bm, CNT_W), lambda i: (i, 0)),
        pl.BlockSpec((bm, CNT_W), lambda i: (i, 0)),
        pl.BlockSpec((bm, w_in), lambda i: (i, 0)),
        pl.BlockSpec((bm, w_in), lambda i: (i, 0)),
        pl.BlockSpec((bm, CNT_W), lambda i: (i, 0)),
        pl.BlockSpec((bm, CNT_W), lambda i: (i, 0)),
        pl.BlockSpec((bm, w_in), lambda i: (i, 0)),
    ]
    in_specs += [pl.BlockSpec(w.shape, lambda i: (0, 0)) for w in ws]
    in_specs += [pl.BlockSpec((1, w.shape[1]), lambda i: (0, 0)) for w in ws]
    if nw == 0:
        out_specs = [pl.BlockSpec((bm, w_in), lambda i: (i, 0))]
        out_shape = [jax.ShapeDtypeStruct((rows, w_in), _f32)]
    else:
        out_specs = [pl.BlockSpec((bm, w.shape[1]), lambda i: (i, 0)) for w in ws]
        out_shape = [jax.ShapeDtypeStruct((rows, w.shape[1]), _f32) for w in ws]
    res = pl.pallas_call(
        bodyf, grid=(grid,), in_specs=in_specs, out_specs=out_specs,
        out_shape=out_shape,
    )(s_a[0], s_a[1], c_a[0], c_a[1], s_b[0], s_b[1], c_b[0], c_b[1], z,
      *ws, *[b.reshape(1, -1) for b in bs])
    return list(res) if isinstance(res, (list, tuple)) else [res]


def _combine_rv(s, c, z, ws, bs, relu):
    w_in = z.shape[1]
    nw = len(ws)

    def bodyf(*refs):
        s0, s1, c0, c1, zr = refs[:5]
        wr = refs[5:5 + nw]
        br = refs[5 + nw:5 + 2 * nw]
        outs = refs[5 + 2 * nw:]
        ssum = (s0[...] + s1[...])[:N_RV]
        cnt = jnp.maximum((c0[...] + c1[...])[:N_RV, :1], 1.0)
        x = ssum / cnt + zr[...]
        if relu:
            x = jnp.maximum(x, 0.0)
        if nw == 0:
            outs[0][...] = x
        else:
            for i in range(nw):
                outs[i][...] = jnp.dot(x, wr[i][...],
                                       preferred_element_type=_f32) + br[i][...]

    in_specs = [
        pl.BlockSpec((NPO, w_in), lambda i: (0, 0)),
        pl.BlockSpec((NPO, w_in), lambda i: (0, 0)),
        pl.BlockSpec((NPO, CNT_W), lambda i: (0, 0)),
        pl.BlockSpec((NPO, CNT_W), lambda i: (0, 0)),
        pl.BlockSpec((N_RV, w_in), lambda i: (0, 0)),
    ]
    in_specs += [pl.BlockSpec(w.shape, lambda i: (0, 0)) for w in ws]
    in_specs += [pl.BlockSpec((1, w.shape[1]), lambda i: (0, 0)) for w in ws]
    if nw == 0:
        out_specs = [pl.BlockSpec((N_RV, w_in), lambda i: (0, 0))]
        out_shape = [jax.ShapeDtypeStruct((N_RV, w_in), _f32)]
    else:
        out_specs = [pl.BlockSpec((N_RV, w.shape[1]), lambda i: (0, 0)) for w in ws]
        out_shape = [jax.ShapeDtypeStruct((N_RV, w.shape[1]), _f32) for w in ws]
    res = pl.pallas_call(
        bodyf, grid=(1,), in_specs=in_specs, out_specs=out_specs,
        out_shape=out_shape,
    )(s[0], s[1], c[0], c[1], z, *ws, *[b.reshape(1, -1) for b in bs])
    return list(res) if isinstance(res, (list, tuple)) else [res]


def kernel(x_room, x_room_virtual, edge_index_rr, edge_index_r_rv, edge_index_rv_r,
           Wn_0_rr, Wr_0_rr, b_0_rr, Wn_0_r_rv, Wr_0_r_rv, b_0_r_rv, Wn_0_rv_r, Wr_0_rv_r, b_0_rv_r,
           Wn_1_rr, Wr_1_rr, b_1_rr, Wn_1_r_rv, Wr_1_r_rv, b_1_r_rv, Wn_1_rv_r, Wr_1_rv_r, b_1_rv_r,
           Wn_2_rr, Wr_2_rr, b_2_rr, Wn_2_r_rv, Wr_2_r_rv, b_2_r_rv, Wn_2_rv_r, Wr_2_rv_r, b_2_rv_r):
    BM = 1000

    srr2, drr2 = _pad_edges(edge_index_rr[0], edge_index_rr[1], RR_NTC)
    sprv2, dprv2 = _pad_edges(edge_index_r_rv[0], edge_index_r_rv[1], PP_NTC)
    srvr2, drvr2 = _pad_edges(edge_index_rv_r[0], edge_index_rv_r[1], PP_NTC)

    c_rr, c_rvr, c_prv = _sc_counts(
        [(drr2, RR_NTC, N_ACC), (drvr2, PP_NTC, N_ACC), (dprv2, PP_NTC, NPO)])

    y_rr0, y_prv0, z_room0 = _mm_multi(
        x_room, [Wn_0_rr, Wn_0_r_rv, Wr_0_rr + Wr_0_rv_r],
        [jnp.zeros_like(b_0_rr), jnp.zeros_like(b_0_rr), b_0_rr + b_0_rv_r], BM)
    y_rvr0, z_rv0 = _mm_multi(
        x_room_virtual, [Wn_0_rv_r, Wr_0_r_rv],
        [jnp.zeros_like(b_0_rv_r), b_0_r_rv], N_RV)

    s_rr0, s_rvr0, s_prv0 = _sc_segsums(
        128,
        [(y_rr0, srr2, drr2, RR_NTC, N_ACC),
         (y_rvr0, srvr2, drvr2, PP_NTC, N_ACC),
         (y_prv0, sprv2, dprv2, PP_NTC, NPO)])

    y_rr1, y_prv1, z_room1 = _combine_room(
        s_rr0, c_rr, s_rvr0, c_rvr, z_room0,
        [Wn_1_rr, Wn_1_r_rv, Wr_1_rr + Wr_1_rv_r],
        [jnp.zeros_like(b_1_rr), jnp.zeros_like(b_1_rr), b_1_rr + b_1_rv_r],
        relu=True, bm=BM)
    y_rvr1, z_rv1 = _combine_rv(
        s_prv0, c_prv, z_rv0, [Wn_1_rv_r, Wr_1_r_rv],
        [jnp.zeros_like(b_1_rv_r), b_1_r_rv], relu=True)

    s_rr1, s_rvr1, s_prv1 = _sc_segsums(
        128,
        [(y_rr1, srr2, drr2, RR_NTC, N_ACC),
         (y_rvr1, srvr2, drvr2, PP_NTC, N_ACC),
         (y_prv1, sprv2, dprv2, PP_NTC, NPO)])

    y_rr2, z_room2 = _combine_room(
        s_rr1, c_rr, s_rvr1, c_rvr, z_room1,
        [Wn_2_rr, Wr_2_rr + Wr_2_rv_r],
        [jnp.zeros_like(b_2_rr), b_2_rr + b_2_rv_r], relu=True, bm=BM)
    (y_rvr2,) = _combine_rv(
        s_prv1, c_prv, z_rv1, [Wn_2_rv_r], [jnp.zeros_like(b_2_rv_r)], relu=True)

    s_rr2, s_rvr2 = _sc_segsums(
        32,
        [(y_rr2, srr2, drr2, RR_NTC, N_ACC),
         (y_rvr2, srvr2, drvr2, PP_NTC, N_ACC)])

    (x3,) = _combine_room(s_rr2, c_rr, s_rvr2, c_rvr, z_room2, [], [],
                          relu=False, bm=BM)

    (s_pool,) = _sc_segsums(32, [(x3, sprv2, dprv2, PP_NTC, NPO)])

    (out,) = _combine_rv(s_pool, c_prv, jnp.zeros((N_RV, 32), _f32), [], [],
                         relu=False)
    return out


# speedup vs baseline: 9.1370x; 2.1692x over previous
import jax
import jax.numpy as jnp
from jax import lax
from jax.experimental import pallas as pl
from jax.experimental.pallas import tpu as pltpu
from jax.experimental.pallas import tpu_sc as plsc

N_ROOM = 10000
N_RV = 1000
E_RR = 320000
E_POOL = 10000

NC = 2
NS = 16
NW = NC * NS
CH = 128

N_ACC = 10112
NPO = 1024
CNT_W = 16
G = 8

RR_NTC = 80
PP_NTC = 3

_f32 = jnp.float32


def _pad_edges(src, dst, ntc):
    e_pad = NW * ntc * CH
    e = src.shape[0]
    src_p = jnp.concatenate([src, jnp.zeros((e_pad - e,), jnp.int32)])
    dst_p = jnp.concatenate([dst, jnp.full((e_pad - e,), N_ROOM, jnp.int32)])
    return src_p.reshape(NW, ntc, CH), dst_p.reshape(NW, ntc, CH)


_SC_PARAMS = pltpu.CompilerParams(use_tc_tiling_on_sc=False)
_ZR = N_ACC // NS


def _sc_counts(passes):
    np_ = len(passes)
    inputs = [p[0] for p in passes]
    inputs.append(jnp.zeros((N_ACC, CNT_W), _f32))
    inputs.append(jnp.ones((CH, CNT_W), _f32))
    out_type = [jax.ShapeDtypeStruct((p[2], CNT_W), _f32)
                for p in passes for _ in range(NC)]
    scratch = [
        pltpu.VMEM((G, CH), jnp.int32),
        pltpu.VMEM((CH, CNT_W), _f32),
        pltpu.VMEM_SHARED((N_ACC, CNT_W), _f32),
        pltpu.SemaphoreType.DMA,
    ]

    def body(*refs):
        it = iter(refs)
        dsts = [next(it) for _ in range(np_)]
        zc = next(it)
        ones_h = next(it)
        couts = [(next(it), next(it)) for _ in range(np_)]
        didx = next(it)
        onesv = next(it)
        accc = next(it)
        sem = next(it)

        cid = lax.axis_index("c")
        sid = lax.axis_index("s")
        wid = cid * NS + sid
        pltpu.sync_copy(ones_h, onesv)

        for p in range(np_):
            ntc = passes[p][1]
            orows = passes[p][2]
            pltpu.sync_copy(zc.at[pl.ds(sid * _ZR, _ZR)],
                            accc.at[pl.ds(sid * _ZR, _ZR)])
            plsc.subcore_barrier()

            dstp = dsts[p]
            g = min(G, ntc)

            @pl.loop(0, ntc // g)
            def _(gi):
                pltpu.sync_copy(dstp.at[wid, pl.ds(gi * g, g)],
                                didx.at[pl.ds(0, g)])
                for i in range(g):
                    pltpu.async_copy(onesv, accc.at[didx.at[i]], sem, add=True)
                for i in range(g):
                    pltpu.make_async_copy(onesv, accc.at[didx.at[i]], sem).wait()

            plsc.subcore_barrier()
            r = orows // NS
            o0, o1 = couts[p]
            for core, oref in enumerate((o0, o1)):
                @pl.when(cid == core)
                def _(oref=oref):
                    pltpu.sync_copy(accc.at[pl.ds(sid * r, r)],
                                    oref.at[pl.ds(sid * r, r)])
            plsc.subcore_barrier()

    mesh = plsc.VectorSubcoreMesh(core_axis_name="c", subcore_axis_name="s")
    fn = pl.kernel(body, out_type=out_type, mesh=mesh, scratch_types=scratch,
                   compiler_params=_SC_PARAMS)
    res = fn(*inputs)
    return [(res[2 * p], res[2 * p + 1]) for p in range(np_)]


def _sc_segsums(width, passes):
    np_ = len(passes)
    inputs = []
    for tab, src3, dst3, ntc, orows in passes:
        inputs += [tab, src3, dst3]
    inputs.append(jnp.zeros((N_ACC, width), _f32))
    out_type = [jax.ShapeDtypeStruct((p[4], width), _f32)
                for p in passes for _ in range(NC)]
    scratch = [
        pltpu.VMEM((G, CH), jnp.int32),
        pltpu.VMEM((G, CH), jnp.int32),
        pltpu.VMEM((CH, width), _f32),
        pltpu.VMEM_SHARED((N_ACC, width), _f32),
        pltpu.SemaphoreType.DMA,
    ]

    def body(*refs):
        it = iter(refs)
        tabs, srcs, dsts = [], [], []
        for _ in range(np_):
            tabs.append(next(it)); srcs.append(next(it)); dsts.append(next(it))
        zw = next(it)
        outs = [(next(it), next(it)) for _ in range(np_)]
        sidx = next(it); didx = next(it)
        rows = next(it)
        acc = next(it)
        sem = next(it)

        cid = lax.axis_index("c")
        sid = lax.axis_index("s")
        wid = cid * NS + sid

        for p in range(np_):
            ntc = passes[p][3]
            orows = passes[p][4]
            pltpu.sync_copy(zw.at[pl.ds(sid * _ZR, _ZR)],
                            acc.at[pl.ds(sid * _ZR, _ZR)])
            plsc.subcore_barrier()

            tab = tabs[p]
            srcp = srcs[p]
            dstp = dsts[p]
            g = min(G, ntc)

            @pl.loop(0, ntc // g)
            def _(gi):
                pltpu.sync_copy(srcp.at[wid, pl.ds(gi * g, g)],
                                sidx.at[pl.ds(0, g)])
                pltpu.sync_copy(dstp.at[wid, pl.ds(gi * g, g)],
                                didx.at[pl.ds(0, g)])
                for k in range(g):
                    pltpu.async_copy(tab.at[sidx.at[k]], rows, sem).wait()
                    pltpu.sync_copy(rows, acc.at[didx.at[k]], add=True)

            plsc.subcore_barrier()
            r = orows // NS
            o0, o1 = outs[p]
            for core, oref in enumerate((o0, o1)):
                @pl.when(cid == core)
                def _(oref=oref):
                    pltpu.sync_copy(acc.at[pl.ds(sid * r, r)],
                                    oref.at[pl.ds(sid * r, r)])
            plsc.subcore_barrier()

    mesh = plsc.VectorSubcoreMesh(core_axis_name="c", subcore_axis_name="s")
    fn = pl.kernel(body, out_type=out_type, mesh=mesh, scratch_types=scratch,
                   compiler_params=_SC_PARAMS)
    res = fn(*inputs)
    if not isinstance(res, (list, tuple)):
        res = [res]
    return [(res[2 * p], res[2 * p + 1]) for p in range(np_)]


def _mm_multi(x, ws, bs, bm):
    rows, k = x.shape
    grid = rows // bm
    nw = len(ws)

    def bodyf(*refs):
        xr = refs[0]
        wr = refs[1:1 + nw]
        br = refs[1 + nw:1 + 2 * nw]
        outs = refs[1 + 2 * nw:]
        xv = xr[...]
        for i in range(nw):
            outs[i][...] = jnp.dot(xv, wr[i][...],
                                   preferred_element_type=_f32) + br[i][...]

    in_specs = [pl.BlockSpec((bm, k), lambda i: (i, 0))]
    in_specs += [pl.BlockSpec(w.shape, lambda i: (0, 0)) for w in ws]
    in_specs += [pl.BlockSpec((1, w.shape[1]), lambda i: (0, 0)) for w in ws]
    out_specs = [pl.BlockSpec((bm, w.shape[1]), lambda i: (i, 0)) for w in ws]
    out_shape = [jax.ShapeDtypeStruct((rows, w.shape[1]), _f32) for w in ws]
    res = pl.pallas_call(
        bodyf, grid=(grid,), in_specs=in_specs, out_specs=out_specs,
        out_shape=out_shape,
    )(x, *ws, *[b.reshape(1, -1) for b in bs])
    return list(res) if isinstance(res, (list, tuple)) else [res]


def _combine_room(s_a, c_a, s_b, c_b, z, ws, bs, relu, bm):
    rows, w_in = z.shape
    grid = rows // bm
    nw = len(ws)

    def bodyf(*refs):
        (sa0, sa1, ca0, ca1, sb0, sb1, cb0, cb1, zr) = refs[:9]
        wr = refs[9:9 + nw]
        br = refs[9 + nw:9 + 2 * nw]
        outs = refs[9 + 2 * nw:]
        cnt_a = jnp.maximum(ca0[:, :1] + ca1[:, :1], 1.0)
        cnt_b = jnp.maximum(cb0[:, :1] + cb1[:, :1], 1.0)
        x = (sa0[...] + sa1[...]) / cnt_a + (sb0[...] + sb1[...]) / cnt_b + zr[...]
        if relu:
            x = jnp.maximum(x, 0.0)
        if nw == 0:
            outs[0][...] = x
        else:
            for i in range(nw):
                outs[i][...] = jnp.dot(x, wr[i][...],
                                       preferred_element_type=_f32) + br[i][...]

    in_specs = [
        pl.BlockSpec((bm, w_in), lambda i: (i, 0)),
        pl.BlockSpec((bm, w_in), lambda i: (i, 0)),
        pl.BlockSpec((bm, CNT_W), lambda i: (i, 0)),
        pl.BlockSpec((bm, CNT_W), lambda i: (i, 0)),
        pl.BlockSpec((bm, w_in), lambda i: (i, 0)),
        pl.BlockSpec((bm, w_in), lambda i: (i, 0)),
        pl.BlockSpec((bm, CNT_W), lambda i: (i, 0)),
        pl.BlockSpec((bm, CNT_W), lambda i: (i, 0)),
        pl.BlockSpec((bm, w_in), lambda i: (i, 0)),
    ]
    in_specs += [pl.BlockSpec(w.shape, lambda i: (0, 0)) for w in ws]
    in_specs += [pl.BlockSpec((1, w.shape[1]), lambda i: (0, 0)) for w in ws]
    if nw == 0:
        out_specs = [pl.BlockSpec((bm, w_in), lambda i: (i, 0))]
        out_shape = [jax.ShapeDtypeStruct((rows, w_in), _f32)]
    else:
        out_specs = [pl.BlockSpec((bm, w.shape[1]), lambda i: (i, 0)) for w in ws]
        out_shape = [jax.ShapeDtypeStruct((rows, w.shape[1]), _f32) for w in ws]
    res = pl.pallas_call(
        bodyf, grid=(grid,), in_specs=in_specs, out_specs=out_specs,
        out_shape=out_shape,
    )(s_a[0], s_a[1], c_a[0], c_a[1], s_b[0], s_b[1], c_b[0], c_b[1], z,
      *ws, *[b.reshape(1, -1) for b in bs])
    return list(res) if isinstance(res, (list, tuple)) else [res]


def _combine_rv(s, c, z, ws, bs, relu):
    w_in = z.shape[1]
    nw = len(ws)

    def bodyf(*refs):
        s0, s1, c0, c1, zr = refs[:5]
        wr = refs[5:5 + nw]
        br = refs[5 + nw:5 + 2 * nw]
        outs = refs[5 + 2 * nw:]
        ssum = (s0[...] + s1[...])[:N_RV]
        cnt = jnp.maximum((c0[...] + c1[...])[:N_RV, :1], 1.0)
        x = ssum / cnt + zr[...]
        if relu:
            x = jnp.maximum(x, 0.0)
        if nw == 0:
            outs[0][...] = x
        else:
            for i in range(nw):
                outs[i][...] = jnp.dot(x, wr[i][...],
                                       preferred_element_type=_f32) + br[i][...]

    in_specs = [
        pl.BlockSpec((NPO, w_in), lambda i: (0, 0)),
        pl.BlockSpec((NPO, w_in), lambda i: (0, 0)),
        pl.BlockSpec((NPO, CNT_W), lambda i: (0, 0)),
        pl.BlockSpec((NPO, CNT_W), lambda i: (0, 0)),
        pl.BlockSpec((N_RV, w_in), lambda i: (0, 0)),
    ]
    in_specs += [pl.BlockSpec(w.shape, lambda i: (0, 0)) for w in ws]
    in_specs += [pl.BlockSpec((1, w.shape[1]), lambda i: (0, 0)) for w in ws]
    if nw == 0:
        out_specs = [pl.BlockSpec((N_RV, w_in), lambda i: (0, 0))]
        out_shape = [jax.ShapeDtypeStruct((N_RV, w_in), _f32)]
    else:
        out_specs = [pl.BlockSpec((N_RV, w.shape[1]), lambda i: (0, 0)) for w in ws]
        out_shape = [jax.ShapeDtypeStruct((N_RV, w.shape[1]), _f32) for w in ws]
    res = pl.pallas_call(
        bodyf, grid=(1,), in_specs=in_specs, out_specs=out_specs,
        out_shape=out_shape,
    )(s[0], s[1], c[0], c[1], z, *ws, *[b.reshape(1, -1) for b in bs])
    return list(res) if isinstance(res, (list, tuple)) else [res]


def kernel(x_room, x_room_virtual, edge_index_rr, edge_index_r_rv, edge_index_rv_r,
           Wn_0_rr, Wr_0_rr, b_0_rr, Wn_0_r_rv, Wr_0_r_rv, b_0_r_rv, Wn_0_rv_r, Wr_0_rv_r, b_0_rv_r,
           Wn_1_rr, Wr_1_rr, b_1_rr, Wn_1_r_rv, Wr_1_r_rv, b_1_r_rv, Wn_1_rv_r, Wr_1_rv_r, b_1_rv_r,
           Wn_2_rr, Wr_2_rr, b_2_rr, Wn_2_r_rv, Wr_2_r_rv, b_2_r_rv, Wn_2_rv_r, Wr_2_rv_r, b_2_rv_r):
    BM = 1000

    srr2, drr2 = _pad_edges(edge_index_rr[0], edge_index_rr[1], RR_NTC)
    sprv2, dprv2 = _pad_edges(edge_index_r_rv[0], edge_index_r_rv[1], PP_NTC)
    srvr2, drvr2 = _pad_edges(edge_index_rv_r[0], edge_index_rv_r[1], PP_NTC)

    c_rr, c_rvr, c_prv = _sc_counts(
        [(drr2, RR_NTC, N_ACC), (drvr2, PP_NTC, N_ACC), (dprv2, PP_NTC, NPO)])

    y_rr0, y_prv0, z_room0 = _mm_multi(
        x_room, [Wn_0_rr, Wn_0_r_rv, Wr_0_rr + Wr_0_rv_r],
        [jnp.zeros_like(b_0_rr), jnp.zeros_like(b_0_rr), b_0_rr + b_0_rv_r], BM)
    y_rvr0, z_rv0 = _mm_multi(
        x_room_virtual, [Wn_0_rv_r, Wr_0_r_rv],
        [jnp.zeros_like(b_0_rv_r), b_0_r_rv], N_RV)

    s_rr0, s_rvr0, s_prv0 = _sc_segsums(
        128,
        [(y_rr0, srr2, drr2, RR_NTC, N_ACC),
         (y_rvr0, srvr2, drvr2, PP_NTC, N_ACC),
         (y_prv0, sprv2, dprv2, PP_NTC, NPO)])

    y_rr1, y_prv1, z_room1 = _combine_room(
        s_rr0, c_rr, s_rvr0, c_rvr, z_room0,
        [Wn_1_rr, Wn_1_r_rv, Wr_1_rr + Wr_1_rv_r],
        [jnp.zeros_like(b_1_rr), jnp.zeros_like(b_1_rr), b_1_rr + b_1_rv_r],
        relu=True, bm=BM)
    y_rvr1, z_rv1 = _combine_rv(
        s_prv0, c_prv, z_rv0, [Wn_1_rv_r, Wr_1_r_rv],
        [jnp.zeros_like(b_1_rv_r), b_1_r_rv], relu=True)

    s_rr1, s_rvr1, s_prv1 = _sc_segsums(
        128,
        [(y_rr1, srr2, drr2, RR_NTC, N_ACC),
         (y_rvr1, srvr2, drvr2, PP_NTC, N_ACC),
         (y_prv1, sprv2, dprv2, PP_NTC, NPO)])

    y_rr2, z_room2 = _combine_room(
        s_rr1, c_rr, s_rvr1, c_rvr, z_room1,
        [Wn_2_rr, Wr_2_rr + Wr_2_rv_r],
        [jnp.zeros_like(b_2_rr), b_2_rr + b_2_rv_r], relu=True, bm=BM)
    (y_rvr2,) = _combine_rv(
        s_prv1, c_prv, z_rv1, [Wn_2_rv_r], [jnp.zeros_like(b_2_rv_r)], relu=True)

    s_rr2, s_rvr2 = _sc_segsums(
        32,
        [(y_rr2, srr2, drr2, RR_NTC, N_ACC),
         (y_rvr2, srvr2, drvr2, PP_NTC, N_ACC)])

    (x3,) = _combine_room(s_rr2, c_rr, s_rvr2, c_rvr, z_room2, [], [],
                          relu=False, bm=BM)

    (s_pool,) = _sc_segsums(32, [(x3, sprv2, dprv2, PP_NTC, NPO)])

    (out,) = _combine_rv(s_pool, c_prv, jnp.zeros((N_RV, 32), _f32), [], [],
                         relu=False)
    return out


# speedup vs baseline: 9.4977x; 1.0395x over previous
import jax
import jax.numpy as jnp
from jax import lax
from jax.experimental import pallas as pl
from jax.experimental.pallas import tpu as pltpu
from jax.experimental.pallas import tpu_sc as plsc

N_ROOM = 10000
N_RV = 1000
E_RR = 320000
E_POOL = 10000

NC = 2
NS = 16
NW = NC * NS
CH = 128

N_ACC = 10112
NPO = 1024
CNT_W = 16
G = 8

RR_NTC = 80
PP_NTC = 3

_f32 = jnp.float32


def _pad_edges(src, dst, ntc):
    e_pad = NW * ntc * CH
    e = src.shape[0]
    src_p = jnp.concatenate([src, jnp.zeros((e_pad - e,), jnp.int32)])
    dst_p = jnp.concatenate([dst, jnp.full((e_pad - e,), N_ROOM, jnp.int32)])
    return src_p.reshape(NW, ntc, CH), dst_p.reshape(NW, ntc, CH)


_SC_PARAMS = pltpu.CompilerParams(use_tc_tiling_on_sc=False)
_ZR = N_ACC // NS


def _sc_segsums(width, passes, with_counts=False):
    np_ = len(passes)
    inputs = []
    for tab, src3, dst3, ntc, orows in passes:
        inputs += [tab, src3, dst3]
    inputs.append(jnp.zeros((N_ACC, width), _f32))
    if with_counts:
        inputs.append(jnp.zeros((N_ACC, CNT_W), _f32))
        inputs.append(jnp.ones((CH, CNT_W), _f32))
    out_type = [jax.ShapeDtypeStruct((p[4], width), _f32)
                for p in passes for _ in range(NC)]
    if with_counts:
        out_type += [jax.ShapeDtypeStruct((p[4], CNT_W), _f32)
                     for p in passes for _ in range(NC)]
    scratch = [
        pltpu.VMEM((G, CH), jnp.int32),
        pltpu.VMEM((G, CH), jnp.int32),
        pltpu.VMEM((CH, width), _f32),
        pltpu.VMEM_SHARED((N_ACC, width), _f32),
        pltpu.SemaphoreType.DMA,
    ]
    if with_counts:
        scratch.append(pltpu.VMEM((CH, CNT_W), _f32))
        scratch.append(pltpu.VMEM_SHARED((N_ACC, CNT_W), _f32))
        scratch.append(pltpu.SemaphoreType.DMA)

    def body(*refs):
        it = iter(refs)
        tabs, srcs, dsts = [], [], []
        for _ in range(np_):
            tabs.append(next(it)); srcs.append(next(it)); dsts.append(next(it))
        zw = next(it)
        if with_counts:
            zc = next(it); ones_h = next(it)
        outs = [(next(it), next(it)) for _ in range(np_)]
        couts = [(next(it), next(it)) for _ in range(np_)] if with_counts else []
        sidx = next(it); didx = next(it)
        rows = next(it)
        acc = next(it)
        sem = next(it)
        if with_counts:
            onesv = next(it); accc = next(it); csem = next(it)

        cid = lax.axis_index("c")
        sid = lax.axis_index("s")
        wid = cid * NS + sid
        if with_counts:
            pltpu.sync_copy(ones_h, onesv)

        for p in range(np_):
            ntc = passes[p][3]
            orows = passes[p][4]
            pltpu.sync_copy(zw.at[pl.ds(sid * _ZR, _ZR)],
                            acc.at[pl.ds(sid * _ZR, _ZR)])
            if with_counts:
                pltpu.sync_copy(zc.at[pl.ds(sid * _ZR, _ZR)],
                                accc.at[pl.ds(sid * _ZR, _ZR)])
            plsc.subcore_barrier()

            tab = tabs[p]
            srcp = srcs[p]
            dstp = dsts[p]
            g = min(G, ntc)

            @pl.loop(0, ntc // g)
            def _(gi):
                pltpu.sync_copy(srcp.at[wid, pl.ds(gi * g, g)],
                                sidx.at[pl.ds(0, g)])
                pltpu.sync_copy(dstp.at[wid, pl.ds(gi * g, g)],
                                didx.at[pl.ds(0, g)])
                for k in range(g):
                    if with_counts:
                        pltpu.async_copy(onesv, accc.at[didx.at[k]], csem,
                                         add=True)
                    pltpu.async_copy(tab.at[sidx.at[k]], rows, sem).wait()
                    pltpu.sync_copy(rows, acc.at[didx.at[k]], add=True)
                if with_counts:
                    for k in range(g):
                        pltpu.make_async_copy(onesv, accc.at[didx.at[k]],
                                              csem).wait()

            plsc.subcore_barrier()
            r = orows // NS
            o0, o1 = outs[p]
            for core, oref in enumerate((o0, o1)):
                @pl.when(cid == core)
                def _(oref=oref, core=core):
                    pltpu.sync_copy(acc.at[pl.ds(sid * r, r)],
                                    oref.at[pl.ds(sid * r, r)])
                    if with_counts:
                        pltpu.sync_copy(accc.at[pl.ds(sid * r, r)],
                                        couts[p][core].at[pl.ds(sid * r, r)])
            plsc.subcore_barrier()

    mesh = plsc.VectorSubcoreMesh(core_axis_name="c", subcore_axis_name="s")
    fn = pl.kernel(body, out_type=out_type, mesh=mesh, scratch_types=scratch,
                   compiler_params=_SC_PARAMS)
    res = fn(*inputs)
    if not isinstance(res, (list, tuple)):
        res = [res]
    sums = [(res[2 * p], res[2 * p + 1]) for p in range(np_)]
    if with_counts:
        counts = [(res[2 * np_ + 2 * p], res[2 * np_ + 2 * p + 1])
                  for p in range(np_)]
        return sums, counts
    return sums


def _mm_multi(x, ws, bs, bm):
    rows, k = x.shape
    grid = rows // bm
    nw = len(ws)

    def bodyf(*refs):
        xr = refs[0]
        wr = refs[1:1 + nw]
        br = refs[1 + nw:1 + 2 * nw]
        outs = refs[1 + 2 * nw:]
        xv = xr[...]
        for i in range(nw):
            outs[i][...] = jnp.dot(xv, wr[i][...],
                                   preferred_element_type=_f32) + br[i][...]

    in_specs = [pl.BlockSpec((bm, k), lambda i: (i, 0))]
    in_specs += [pl.BlockSpec(w.shape, lambda i: (0, 0)) for w in ws]
    in_specs += [pl.BlockSpec((1, w.shape[1]), lambda i: (0, 0)) for w in ws]
    out_specs = [pl.BlockSpec((bm, w.shape[1]), lambda i: (i, 0)) for w in ws]
    out_shape = [jax.ShapeDtypeStruct((rows, w.shape[1]), _f32) for w in ws]
    res = pl.pallas_call(
        bodyf, grid=(grid,), in_specs=in_specs, out_specs=out_specs,
        out_shape=out_shape,
    )(x, *ws, *[b.reshape(1, -1) for b in bs])
    return list(res) if isinstance(res, (list, tuple)) else [res]


def _combine_room(s_a, c_a, s_b, c_b, z, ws, bs, relu, bm):
    rows, w_in = z.shape
    grid = rows // bm
    nw = len(ws)

    def bodyf(*refs):
        (sa0, sa1, ca0, ca1, sb0, sb1, cb0, cb1, zr) = refs[:9]
        wr = refs[9:9 + nw]
        br = refs[9 + nw:9 + 2 * nw]
        outs = refs[9 + 2 * nw:]
        cnt_a = jnp.maximum(ca0[:, :1] + ca1[:, :1], 1.0)
        cnt_b = jnp.maximum(cb0[:, :1] + cb1[:, :1], 1.0)
        x = (sa0[...] + sa1[...]) / cnt_a + (sb0[...] + sb1[...]) / cnt_b + zr[...]
        if relu:
            x = jnp.maximum(x, 0.0)
        if nw == 0:
            outs[0][...] = x
        else:
            for i in range(nw):
                outs[i][...] = jnp.dot(x, wr[i][...],
                                       preferred_element_type=_f32) + br[i][...]

    in_specs = [
        pl.BlockSpec((bm, w_in), lambda i: (i, 0)),
        pl.BlockSpec((bm, w_in), lambda i: (i, 0)),
        pl.BlockSpec((bm, CNT_W), lambda i: (i, 0)),
        pl.BlockSpec((bm, CNT_W), lambda i: (i, 0)),
        pl.BlockSpec((bm, w_in), lambda i: (i, 0)),
        pl.BlockSpec((bm, w_in), lambda i: (i, 0)),
        pl.BlockSpec((bm, CNT_W), lambda i: (i, 0)),
        pl.BlockSpec((bm, CNT_W), lambda i: (i, 0)),
        pl.BlockSpec((bm, w_in), lambda i: (i, 0)),
    ]
    in_specs += [pl.BlockSpec(w.shape, lambda i: (0, 0)) for w in ws]
    in_specs += [pl.BlockSpec((1, w.shape[1]), lambda i: (0, 0)) for w in ws]
    if nw == 0:
        out_specs = [pl.BlockSpec((bm, w_in), lambda i: (i, 0))]
        out_shape = [jax.ShapeDtypeStruct((rows, w_in), _f32)]
    else:
        out_specs = [pl.BlockSpec((bm, w.shape[1]), lambda i: (i, 0)) for w in ws]
        out_shape = [jax.ShapeDtypeStruct((rows, w.shape[1]), _f32) for w in ws]
    res = pl.pallas_call(
        bodyf, grid=(grid,), in_specs=in_specs, out_specs=out_specs,
        out_shape=out_shape,
    )(s_a[0], s_a[1], c_a[0], c_a[1], s_b[0], s_b[1], c_b[0], c_b[1], z,
      *ws, *[b.reshape(1, -1) for b in bs])
    return list(res) if isinstance(res, (list, tuple)) else [res]


def _combine_rv(s, c, z, ws, bs, relu):
    w_in = z.shape[1]
    nw = len(ws)

    def bodyf(*refs):
        s0, s1, c0, c1, zr = refs[:5]
        wr = refs[5:5 + nw]
        br = refs[5 + nw:5 + 2 * nw]
        outs = refs[5 + 2 * nw:]
        ssum = (s0[...] + s1[...])[:N_RV]
        cnt = jnp.maximum((c0[...] + c1[...])[:N_RV, :1], 1.0)
        x = ssum / cnt + zr[...]
        if relu:
            x = jnp.maximum(x, 0.0)
        if nw == 0:
            outs[0][...] = x
        else:
            for i in range(nw):
                outs[i][...] = jnp.dot(x, wr[i][...],
                                       preferred_element_type=_f32) + br[i][...]

    in_specs = [
        pl.BlockSpec((NPO, w_in), lambda i: (0, 0)),
        pl.BlockSpec((NPO, w_in), lambda i: (0, 0)),
        pl.BlockSpec((NPO, CNT_W), lambda i: (0, 0)),
        pl.BlockSpec((NPO, CNT_W), lambda i: (0, 0)),
        pl.BlockSpec((N_RV, w_in), lambda i: (0, 0)),
    ]
    in_specs += [pl.BlockSpec(w.shape, lambda i: (0, 0)) for w in ws]
    in_specs += [pl.BlockSpec((1, w.shape[1]), lambda i: (0, 0)) for w in ws]
    if nw == 0:
        out_specs = [pl.BlockSpec((N_RV, w_in), lambda i: (0, 0))]
        out_shape = [jax.ShapeDtypeStruct((N_RV, w_in), _f32)]
    else:
        out_specs = [pl.BlockSpec((N_RV, w.shape[1]), lambda i: (0, 0)) for w in ws]
        out_shape = [jax.ShapeDtypeStruct((N_RV, w.shape[1]), _f32) for w in ws]
    res = pl.pallas_call(
        bodyf, grid=(1,), in_specs=in_specs, out_specs=out_specs,
        out_shape=out_shape,
    )(s[0], s[1], c[0], c[1], z, *ws, *[b.reshape(1, -1) for b in bs])
    return list(res) if isinstance(res, (list, tuple)) else [res]


def kernel(x_room, x_room_virtual, edge_index_rr, edge_index_r_rv, edge_index_rv_r,
           Wn_0_rr, Wr_0_rr, b_0_rr, Wn_0_r_rv, Wr_0_r_rv, b_0_r_rv, Wn_0_rv_r, Wr_0_rv_r, b_0_rv_r,
           Wn_1_rr, Wr_1_rr, b_1_rr, Wn_1_r_rv, Wr_1_r_rv, b_1_r_rv, Wn_1_rv_r, Wr_1_rv_r, b_1_rv_r,
           Wn_2_rr, Wr_2_rr, b_2_rr, Wn_2_r_rv, Wr_2_r_rv, b_2_r_rv, Wn_2_rv_r, Wr_2_rv_r, b_2_rv_r):
    BM = 1000

    srr2, drr2 = _pad_edges(edge_index_rr[0], edge_index_rr[1], RR_NTC)
    sprv2, dprv2 = _pad_edges(edge_index_r_rv[0], edge_index_r_rv[1], PP_NTC)
    srvr2, drvr2 = _pad_edges(edge_index_rv_r[0], edge_index_rv_r[1], PP_NTC)

    y_rr0, y_prv0, z_room0 = _mm_multi(
        x_room, [Wn_0_rr, Wn_0_r_rv, Wr_0_rr + Wr_0_rv_r],
        [jnp.zeros_like(b_0_rr), jnp.zeros_like(b_0_rr), b_0_rr + b_0_rv_r], BM)
    y_rvr0, z_rv0 = _mm_multi(
        x_room_virtual, [Wn_0_rv_r, Wr_0_r_rv],
        [jnp.zeros_like(b_0_rv_r), b_0_r_rv], N_RV)

    (s_rr0, s_rvr0, s_prv0), (c_rr, c_rvr, c_prv) = _sc_segsums(
        128,
        [(y_rr0, srr2, drr2, RR_NTC, N_ACC),
         (y_rvr0, srvr2, drvr2, PP_NTC, N_ACC),
         (y_prv0, sprv2, dprv2, PP_NTC, NPO)],
        with_counts=True)

    y_rr1, y_prv1, z_room1 = _combine_room(
        s_rr0, c_rr, s_rvr0, c_rvr, z_room0,
        [Wn_1_rr, Wn_1_r_rv, Wr_1_rr + Wr_1_rv_r],
        [jnp.zeros_like(b_1_rr), jnp.zeros_like(b_1_rr), b_1_rr + b_1_rv_r],
        relu=True, bm=BM)
    y_rvr1, z_rv1 = _combine_rv(
        s_prv0, c_prv, z_rv0, [Wn_1_rv_r, Wr_1_r_rv],
        [jnp.zeros_like(b_1_rv_r), b_1_r_rv], relu=True)

    s_rr1, s_rvr1, s_prv1 = _sc_segsums(
        128,
        [(y_rr1, srr2, drr2, RR_NTC, N_ACC),
         (y_rvr1, srvr2, drvr2, PP_NTC, N_ACC),
         (y_prv1, sprv2, dprv2, PP_NTC, NPO)])

    y_rr2, z_room2 = _combine_room(
        s_rr1, c_rr, s_rvr1, c_rvr, z_room1,
        [Wn_2_rr, Wr_2_rr + Wr_2_rv_r],
        [jnp.zeros_like(b_2_rr), b_2_rr + b_2_rv_r], relu=True, bm=BM)
    (y_rvr2,) = _combine_rv(
        s_prv1, c_prv, z_rv1, [Wn_2_rv_r], [jnp.zeros_like(b_2_rv_r)], relu=True)

    s_rr2, s_rvr2 = _sc_segsums(
        32,
        [(y_rr2, srr2, drr2, RR_NTC, N_ACC),
         (y_rvr2, srvr2, drvr2, PP_NTC, N_ACC)])

    (x3,) = _combine_room(s_rr2, c_rr, s_rvr2, c_rvr, z_room2, [], [],
                          relu=False, bm=BM)

    (s_pool,) = _sc_segsums(32, [(x3, sprv2, dprv2, PP_NTC, NPO)])

    (out,) = _combine_rv(s_pool, c_prv, jnp.zeros((N_RV, 32), _f32), [], [],
                         relu=False)
    return out
